# Initial kernel scaffold; baseline (speedup 1.0000x reference)
import jax
import jax.numpy as jnp
from jax.experimental import pallas as pl


def kernel(x, lengths):
    raise NotImplementedError("write your pallas kernel here")


# speedup vs baseline: 1.0333x; 1.0333x over previous
import functools

import jax
import jax.numpy as jnp
from jax import lax
from jax.experimental import pallas as pl
from jax.experimental.pallas import tpu as pltpu
from jax.experimental.pallas import tpu_sc as plsc

_LANES = 16


def _last_token_gather(T, x_hbm, len_hbm, out_hbm, len_v, idx_v, rows_v, sem):
    B, C = out_hbm.shape
    cid = lax.axis_index("c")
    sid = lax.axis_index("s")

    @pl.when(jnp.logical_and(cid == 0, sid == 0))
    def _():
        pltpu.sync_copy(len_hbm, len_v)
        lane = lax.iota(jnp.int32, _LANES)
        gi = jnp.maximum(len_v[...] - 1, 0) + lane * T
        idx_v[...] = jnp.where(lane < B, gi, 0)
        pltpu.async_copy(x_hbm.at[idx_v], rows_v, sem).wait()
        pltpu.sync_copy(rows_v.at[pl.ds(0, B)], out_hbm)


def kernel(x, lengths):
    B, T, C = x.shape
    x_flat = x.reshape(B * T, C)
    len_pad = jnp.zeros((_LANES,), jnp.int32).at[:B].set(lengths.astype(jnp.int32))

    mesh = plsc.VectorSubcoreMesh(core_axis_name="c", subcore_axis_name="s")
    run = functools.partial(
        pl.kernel,
        out_type=jax.ShapeDtypeStruct((B, C), x.dtype),
        mesh=mesh,
        scratch_types=[
            pltpu.VMEM((_LANES,), jnp.int32),
            pltpu.VMEM((_LANES,), jnp.int32),
            pltpu.VMEM((_LANES, C), x.dtype),
            pltpu.SemaphoreType.DMA,
        ],
    )(functools.partial(_last_token_gather, T))
    return run(x_flat, len_pad)
